# Initial kernel scaffold; baseline (speedup 1.0000x reference)
import functools

import numpy as np
import jax
import jax.numpy as jnp
from jax import lax
from jax.experimental import pallas as pl
from jax.experimental.pallas import tpu as pltpu
from jax.experimental.pallas import tpu_sc as plsc

VOCAB = 100000
D = 64
B = 4096
L = 200
N = B * L

NC = 2
NS = 16
NW = NC * NS
ROWS_PER_W = B // NW


def _positional_encoding() -> np.ndarray:
    pos = np.arange(L, dtype=np.float64)[:, None]
    idx = np.arange(D, dtype=np.float64)[None, :]
    inner = pos / np.power(10000.0, 2.0 * idx / D)
    even = (np.arange(D)[None, :] % 2) == 0
    pe = np.where(even, np.sin(inner), np.cos(inner))
    return pe.astype(np.float32)


_PE = _positional_encoding()


def _sc_embed(x_flat, table, pe):
    mesh = plsc.VectorSubcoreMesh(
        core_axis_name="c", subcore_axis_name="s", num_cores=NC, num_subcores=NS
    )

    @functools.partial(
        pl.kernel,
        out_type=jax.ShapeDtypeStruct((N, D), jnp.float32),
        mesh=mesh,
        scratch_types=[
            pltpu.VMEM((L,), jnp.int32),
            pltpu.VMEM((L, D), jnp.float32),
            pltpu.VMEM((L, D), jnp.float32),
            pltpu.SemaphoreType.DMA,
        ],
    )
    def k(x_hbm, tab_hbm, pe_hbm, out_hbm, idx_v, rows_v, pe_v, sem):
        wid = lax.axis_index("s") * NC + lax.axis_index("c")
        pltpu.sync_copy(pe_hbm, pe_v)

        def chunk(ci, carry):
            start = (wid * ROWS_PER_W + ci) * L
            pltpu.sync_copy(x_hbm.at[pl.ds(start, L)], idx_v)
            pltpu.async_copy(tab_hbm.at[idx_v], rows_v, sem).wait()

            def row(r, c2):
                for c in range(D // 16):
                    sl = pl.ds(c * 16, 16)
                    rows_v[r, sl] = (rows_v[r, sl] + pe_v[r, sl]) * 8.0
                return c2

            lax.fori_loop(0, L, row, 0)
            pltpu.sync_copy(rows_v, out_hbm.at[pl.ds(start, L)])
            return carry

        lax.fori_loop(0, ROWS_PER_W, chunk, 0)

    return k(x_flat, table, pe)


def kernel(x, table):
    pe = jnp.asarray(_PE)
    out = _sc_embed(x.reshape(N), table, pe)
    return out.reshape(B, L, D)


# speedup vs baseline: 3.0329x; 3.0329x over previous
import functools

import numpy as np
import jax
import jax.numpy as jnp
from jax import lax
from jax.experimental import pallas as pl
from jax.experimental.pallas import tpu as pltpu
from jax.experimental.pallas import tpu_sc as plsc

VOCAB = 100000
D = 64
B = 4096
L = 200
N = B * L

NC = 2
NS = 16
NW = NC * NS
ROWS_PER_W = B // NW


def _positional_encoding() -> np.ndarray:
    pos = np.arange(L, dtype=np.float64)[:, None]
    idx = np.arange(D, dtype=np.float64)[None, :]
    inner = pos / np.power(10000.0, 2.0 * idx / D)
    even = (np.arange(D)[None, :] % 2) == 0
    pe = np.where(even, np.sin(inner), np.cos(inner))
    return pe.astype(np.float32)


_PE = _positional_encoding()


def _sc_embed(x_flat, table, pe):
    mesh = plsc.VectorSubcoreMesh(
        core_axis_name="c", subcore_axis_name="s", num_cores=NC, num_subcores=NS
    )

    @functools.partial(
        pl.kernel,
        out_type=jax.ShapeDtypeStruct((N, D), jnp.float32),
        mesh=mesh,
        scratch_types=[
            pltpu.VMEM((L,), jnp.int32),
            pltpu.VMEM((L, D), jnp.float32),
            pltpu.VMEM((L, D), jnp.float32),
            pltpu.SemaphoreType.DMA,
        ],
        compiler_params=pltpu.CompilerParams(use_tc_tiling_on_sc=False),
    )
    def k(x_hbm, tab_hbm, pe_hbm, out_hbm, idx_v, rows_v, pe_v, sem):
        wid = lax.axis_index("s") * NC + lax.axis_index("c")
        pltpu.sync_copy(pe_hbm, pe_v)

        def chunk(ci, carry):
            start = (wid * ROWS_PER_W + ci) * L
            pltpu.sync_copy(x_hbm.at[pl.ds(start, L)], idx_v)
            pltpu.async_copy(tab_hbm.at[idx_v], rows_v, sem).wait()

            def row(r, c2):
                for c in range(D // 16):
                    sl = pl.ds(c * 16, 16)
                    rows_v[r, sl] = (rows_v[r, sl] + pe_v[r, sl]) * 8.0
                return c2

            lax.fori_loop(0, L, row, 0)
            pltpu.sync_copy(rows_v, out_hbm.at[pl.ds(start, L)])
            return carry

        lax.fori_loop(0, ROWS_PER_W, chunk, 0)

    return k(x_flat, table, pe)


def kernel(x, table):
    pe = jnp.asarray(_PE)
    out = _sc_embed(x.reshape(N), table, pe)
    return out.reshape(B, L, D)


# speedup vs baseline: 4.2274x; 1.3938x over previous
import functools

import numpy as np
import jax
import jax.numpy as jnp
from jax import lax
from jax.experimental import pallas as pl
from jax.experimental.pallas import tpu as pltpu
from jax.experimental.pallas import tpu_sc as plsc

VOCAB = 100000
D = 64
B = 4096
L = 200
N = B * L

NC = 2
NS = 16
NW = NC * NS
ROWS_PER_W = B // NW

CH = 2
CHR = CH * L
NCHUNK = ROWS_PER_W // CH
NBUF = 4


def _positional_encoding() -> np.ndarray:
    pos = np.arange(L, dtype=np.float64)[:, None]
    idx = np.arange(D, dtype=np.float64)[None, :]
    inner = pos / np.power(10000.0, 2.0 * idx / D)
    even = (np.arange(D)[None, :] % 2) == 0
    pe = np.where(even, np.sin(inner), np.cos(inner))
    return pe.astype(np.float32)


_PE = _positional_encoding()


def _sc_embed(x_flat, table, pe):
    mesh = plsc.VectorSubcoreMesh(
        core_axis_name="c", subcore_axis_name="s", num_cores=NC, num_subcores=NS
    )

    @functools.partial(
        pl.kernel,
        out_type=jax.ShapeDtypeStruct((N, D), jnp.float32),
        mesh=mesh,
        scratch_types=[
            pltpu.VMEM((L, D), jnp.float32),
            [pltpu.VMEM((CHR,), jnp.int32) for _ in range(NBUF)],
            [pltpu.VMEM((CHR, D), jnp.float32) for _ in range(NBUF)],
            [pltpu.SemaphoreType.DMA for _ in range(NBUF)],
            [pltpu.SemaphoreType.DMA for _ in range(NBUF)],
        ],
        compiler_params=pltpu.CompilerParams(use_tc_tiling_on_sc=False),
    )
    def k(x_hbm, tab_hbm, pe_hbm, out_hbm, pe_v, idx_v, rows_v, gsem, wsem):
        wid = lax.axis_index("s") * NC + lax.axis_index("c")
        base = wid * ROWS_PER_W * L

        pltpu.sync_copy(pe_hbm, pe_v)

        def start_gather(b, ci):
            pltpu.sync_copy(x_hbm.at[pl.ds(base + ci * CHR, CHR)], idx_v[b])
            pltpu.async_copy(tab_hbm.at[idx_v[b]], rows_v[b], gsem[b])

        for b in range(NBUF - 1):
            start_gather(b, b)

        def compute(b):
            rows = rows_v[b]

            def body(r_it, carry):
                r0 = r_it * 4
                for dr in range(4):
                    r = r0 + dr
                    for c in range(D // 16):
                        sl = pl.ds(c * 16, 16)
                        pe_val = pe_v[r, sl]
                        for s in range(CH):
                            rr = s * L + r
                            rows[rr, sl] = (rows[rr, sl] + pe_val) * 8.0
                return carry

            lax.fori_loop(0, L // 4, body, 0)

        def step(it, carry):
            for b in range(NBUF):
                ci = it * NBUF + b
                pltpu.make_async_copy(tab_hbm.at[idx_v[b]], rows_v[b], gsem[b]).wait()
                compute(b)
                out_slice = out_hbm.at[pl.ds(base + ci * CHR, CHR)]
                pltpu.async_copy(rows_v[b], out_slice, wsem[b])

                nci = ci + NBUF - 1
                pb = (b + NBUF - 1) % NBUF

                @pl.when(nci < NCHUNK)
                def _prep():
                    @pl.when(ci >= 1)
                    def _drain_prev_write():
                        prev_out = out_hbm.at[pl.ds(base + (ci - 1) * CHR, CHR)]
                        pltpu.make_async_copy(rows_v[pb], prev_out, wsem[pb]).wait()

                    start_gather(pb, nci)

            return carry

        lax.fori_loop(0, NCHUNK // NBUF, step, 0)

        for b in range(NBUF):
            ci = NCHUNK - NBUF + b
            out_slice = out_hbm.at[pl.ds(base + ci * CHR, CHR)]
            pltpu.make_async_copy(rows_v[b], out_slice, wsem[b]).wait()

    return k(x_flat, table, pe)


def kernel(x, table):
    pe = jnp.asarray(_PE)
    out = _sc_embed(x.reshape(N), table, pe)
    return out.reshape(B, L, D)
